# baseline (speedup 1.0000x reference)
import functools

import jax
import jax.numpy as jnp
from jax import lax
from jax.experimental import pallas as pl
from jax.experimental.pallas import tpu as pltpu
from jax.experimental.pallas import tpu_sc as plsc

NUM_EXPERTS = 4
TOP_K = 2
D_MODEL = 512
D_FF = 2048
N_TOKENS = 4096
SINKHORN_ITERS = 30

N_GROUPS = 6
BT = 128
NC, NS = 2, 16
NW = NC * NS
P_DISP = 5120
N_TILES = P_DISP // BT

EA_TAB = (0, 0, 0, 1, 1, 2)
EB_TAB = (1, 2, 3, 2, 3, 3)


def _router_body(x_ref, rw_ref, pos_ref, wa_ref, wb_ref, tea_ref, teb_ref,
                 rm_ref):
    lt = lax.dot_general(
        rw_ref[...], x_ref[...],
        (((0,), (1,)), ((), ())),
        preferred_element_type=jnp.float32,
    )

    cost = jnp.exp(lt)
    n0 = jnp.float32(N_TOKENS)
    n1 = jnp.float32(NUM_EXPERTS)
    eps = jnp.float32(1e-8)

    def body(_, carry):
        d0, d1 = carry
        d0 = (1.0 / n0) / (jnp.sum(d1 * cost, axis=0, keepdims=True) + eps)
        d1 = (1.0 / n1) / (jnp.sum(d0 * cost, axis=1, keepdims=True) + eps)
        return d0, d1

    d0 = jnp.ones((1, N_TOKENS), jnp.float32)
    d1 = jnp.ones((NUM_EXPERTS, 1), jnp.float32)
    d0, d1 = lax.fori_loop(0, SINKHORN_ITERS, body, (d0, d1))
    s = d1 * cost * d0

    erow = lax.broadcasted_iota(jnp.int32, (NUM_EXPERTS, N_TOKENS), 0)

    m1 = jnp.max(s, axis=0, keepdims=True)
    i1 = jnp.min(jnp.where(s == m1, erow, NUM_EXPERTS), axis=0, keepdims=True)
    masked = jnp.where(erow == i1, float("-inf"), s)
    m2 = jnp.max(masked, axis=0, keepdims=True)
    i2 = jnp.min(jnp.where(masked == m2, erow, NUM_EXPERTS), axis=0,
                 keepdims=True)

    mx = jnp.max(lt, axis=0, keepdims=True)
    p = jnp.exp(lt - mx)
    p = p / jnp.sum(p, axis=0, keepdims=True)
    s1 = jnp.sum(p * (erow == i1).astype(jnp.float32), axis=0, keepdims=True)
    s2 = jnp.sum(p * (erow == i2).astype(jnp.float32), axis=0, keepdims=True)

    a = jnp.minimum(i1, i2)
    b = jnp.maximum(i1, i2)
    first_is_a = i1 < i2
    wa = jnp.where(first_is_a, s1, s2)
    wb = jnp.where(first_is_a, s2, s1)
    g = a * (7 - a) // 2 + b - a - 1

    grow = lax.broadcasted_iota(jnp.int32, (N_GROUPS, N_TOKENS), 0)
    onehot = (grow == g).astype(jnp.float32)

    blk = 512
    r_iota = lax.broadcasted_iota(jnp.int32, (blk, blk), 0)
    c_iota = lax.broadcasted_iota(jnp.int32, (blk, blk), 1)
    u_strict = (r_iota < c_iota).astype(jnp.float32)
    carry = jnp.zeros((N_GROUPS, 1), jnp.float32)
    rank_parts = []
    for bi in range(N_TOKENS // blk):
        ob = onehot[:, bi * blk:(bi + 1) * blk]
        r6 = lax.dot_general(ob, u_strict, (((1,), (0,)), ((), ())),
                             preferred_element_type=jnp.float32) + carry
        rank_parts.append(jnp.sum(ob * r6, axis=0, keepdims=True))
        carry = carry + jnp.sum(ob, axis=1, keepdims=True)
    rank = jnp.concatenate(rank_parts, axis=1)

    counts = carry
    cap = ((counts.astype(jnp.int32) + (BT - 1)) // BT) * BT
    l6r = lax.broadcasted_iota(jnp.int32, (N_GROUPS, N_GROUPS), 0)
    l6c = lax.broadcasted_iota(jnp.int32, (N_GROUPS, N_GROUPS), 1)
    l_strict = (l6c < l6r).astype(jnp.float32)
    off = lax.dot_general(l_strict, cap.astype(jnp.float32),
                          (((1,), (0,)), ((), ())),
                          preferred_element_type=jnp.float32)

    pos = rank + jnp.sum(onehot * off, axis=0, keepdims=True)
    pos_ref[...] = pos.astype(jnp.int32)
    wa_ref[...] = wa
    wb_ref[...] = wb

    ntile_pad = tea_ref.shape[1]
    trow = lax.broadcasted_iota(jnp.int32, (N_GROUPS, ntile_pad), 1)
    t_start = (off.astype(jnp.int32)) // BT
    t_end = t_start + cap // BT
    inr = ((trow >= t_start) & (trow < t_end)).astype(jnp.int32)
    grow6 = lax.broadcasted_iota(jnp.int32, (N_GROUPS, 1), 0)
    ea_col = jnp.where(grow6 < 3, 0, jnp.where(grow6 < 5, 1, 2))
    eb_col = jnp.where(grow6 == 0, 1,
                       jnp.where(grow6 == 1, 2,
                                 jnp.where(grow6 == 2, 3,
                                           jnp.where(grow6 == 3, 2, 3))))
    tea_ref[...] = jnp.sum(inr * ea_col, axis=0, keepdims=True)
    teb_ref[...] = jnp.sum(inr * eb_col, axis=0, keepdims=True)

    q_iota = lax.broadcasted_iota(jnp.int32, (N_GROUPS, P_DISP), 1)
    off_i = off.astype(jnp.int32)
    end_i = off_i + counts.astype(jnp.int32)
    rm_ref[...] = jnp.sum(((q_iota >= off_i) & (q_iota < end_i))
                          .astype(jnp.int32), axis=0, keepdims=True)


DW = D_MODEL // 2


def _dispatch_body(pos_hbm, wa_hbm, wb_hbm, xp_hbm,
                   xs_hbm, dwa_hbm, dwb_hbm, dtok_hbm,
                   pos_v, wa_v, wb_v, toks_v, xrow_v, sem):
    wid = lax.axis_index("s") * NC + lax.axis_index("c")
    bpw = N_TOKENS // NW
    base = wid * bpw
    pltpu.sync_copy(pos_hbm.at[pl.ds(base, bpw)], pos_v)
    pltpu.sync_copy(wa_hbm.at[pl.ds(base, bpw)], wa_v)
    pltpu.sync_copy(wb_hbm.at[pl.ds(base, bpw)], wb_v)
    pltpu.sync_copy(xp_hbm.at[pl.ds(base, bpw)], xrow_v)

    def iota_body(q, carry):
        toks_v[pl.ds(q * 16, 16)] = base + q * 16 + lax.iota(jnp.int32, 16)
        return carry

    lax.fori_loop(0, bpw // 16, iota_body, 0)

    d1 = pltpu.async_copy(xrow_v, xs_hbm.at[pos_v], sem)
    d2 = pltpu.async_copy(toks_v, dtok_hbm.at[pos_v], sem)
    d3 = pltpu.async_copy(wa_v, dwa_hbm.at[pos_v], sem)
    d4 = pltpu.async_copy(wb_v, dwb_hbm.at[pos_v], sem)
    d1.wait()
    d2.wait()
    d3.wait()
    d4.wait()


def _combine_body(dtok_hbm, rm_hbm, yc_hbm, oext_hbm,
                  tgt_v, rm_v, rows_v, sem):
    wid = lax.axis_index("s") * NC + lax.axis_index("c")
    bpw = P_DISP // NW
    base = wid * bpw
    pltpu.sync_copy(dtok_hbm.at[pl.ds(base, bpw)], tgt_v)
    pltpu.sync_copy(rm_hbm.at[pl.ds(base, bpw)], rm_v)
    pltpu.sync_copy(yc_hbm.at[pl.ds(base, bpw)], rows_v)

    def fix(q, carry):
        sl = pl.ds(q * 16, 16)
        qid = base + q * 16 + lax.iota(jnp.int32, 16)
        trash = N_TOKENS + jnp.bitwise_and(qid, 1023)
        tgt_v[sl] = jnp.where(rm_v[sl] > 0, tgt_v[sl], trash)
        return carry

    lax.fori_loop(0, bpw // 16, fix, 0)
    pltpu.async_copy(rows_v, oext_hbm.at[tgt_v], sem).wait()


def _gemm_body(tea_ref, teb_ref, xs_ref, dwa_ref, dwb_ref, w1_ref, w2_ref,
               yc_ref):
    i = pl.program_id(0)
    ea = tea_ref[i]
    eb = teb_ref[i]
    xb = xs_ref[...]

    def ffn(e):
        h = jnp.dot(xb, w1_ref[e].astype(jnp.bfloat16),
                    preferred_element_type=jnp.float32)
        h = h * jax.nn.sigmoid(h)
        return jnp.dot(h.astype(jnp.bfloat16), w2_ref[e].astype(jnp.bfloat16),
                       preferred_element_type=jnp.float32)

    y = ffn(ea) * dwa_ref[...] + ffn(eb) * dwb_ref[...]
    yc_ref[...] = y.astype(jnp.bfloat16)


_SC_MESH = dict(core_axis_name="c", subcore_axis_name="s",
                num_cores=NC, num_subcores=NS)


@jax.jit
def kernel(x, router_w, w1, w2):
    f32 = jnp.float32
    i32 = jnp.int32
    pos2, wa2, wb2, tea2, teb2, rm2 = pl.pallas_call(
        _router_body,
        out_shape=[
            jax.ShapeDtypeStruct((1, N_TOKENS), i32),
            jax.ShapeDtypeStruct((1, N_TOKENS), f32),
            jax.ShapeDtypeStruct((1, N_TOKENS), f32),
            jax.ShapeDtypeStruct((1, 64), i32),
            jax.ShapeDtypeStruct((1, 64), i32),
            jax.ShapeDtypeStruct((1, P_DISP), i32),
        ],
    )(x, router_w)
    pos = pos2.reshape(N_TOKENS)
    wa = wa2.reshape(N_TOKENS)
    wb = wb2.reshape(N_TOKENS)
    tea = tea2.reshape(64)
    teb = teb2.reshape(64)
    rm = rm2.reshape(P_DISP)

    xp = lax.bitcast_convert_type(
        x.astype(jnp.bfloat16).reshape(N_TOKENS, DW, 2), i32)

    dispatch = pl.kernel(
        _dispatch_body,
        out_type=[
            jax.ShapeDtypeStruct((P_DISP, DW), i32),
            jax.ShapeDtypeStruct((P_DISP,), f32),
            jax.ShapeDtypeStruct((P_DISP,), f32),
            jax.ShapeDtypeStruct((P_DISP,), i32),
        ],
        mesh=plsc.VectorSubcoreMesh(**_SC_MESH),
        scratch_types=[
            pltpu.VMEM((N_TOKENS // NW,), i32),
            pltpu.VMEM((N_TOKENS // NW,), f32),
            pltpu.VMEM((N_TOKENS // NW,), f32),
            pltpu.VMEM((N_TOKENS // NW,), i32),
            pltpu.VMEM((N_TOKENS // NW, DW), i32),
            pltpu.SemaphoreType.DMA,
        ],
    )
    xs, dwa, dwb, dtok = dispatch(pos, wa, wb, xp)

    yc = pl.pallas_call(
        _gemm_body,
        grid=(N_TILES,),
        in_specs=[
            pl.BlockSpec(memory_space=pltpu.SMEM),
            pl.BlockSpec(memory_space=pltpu.SMEM),
            pl.BlockSpec((BT, D_MODEL), lambda i: (i, 0)),
            pl.BlockSpec((BT, 1), lambda i: (i, 0)),
            pl.BlockSpec((BT, 1), lambda i: (i, 0)),
            pl.BlockSpec((NUM_EXPERTS, D_MODEL, D_FF), lambda i: (0, 0, 0)),
            pl.BlockSpec((NUM_EXPERTS, D_FF, D_MODEL), lambda i: (0, 0, 0)),
        ],
        out_specs=pl.BlockSpec((BT, D_MODEL), lambda i: (i, 0)),
        out_shape=jax.ShapeDtypeStruct((P_DISP, D_MODEL), jnp.bfloat16),
        compiler_params=pltpu.CompilerParams(
            dimension_semantics=("arbitrary",),
        ),
    )(tea, teb,
      lax.bitcast_convert_type(xs, jnp.bfloat16).reshape(P_DISP, D_MODEL),
      dwa.reshape(P_DISP, 1), dwb.reshape(P_DISP, 1), w1, w2)
    yc = lax.bitcast_convert_type(yc.reshape(P_DISP, DW, 2), i32)

    combine = pl.kernel(
        _combine_body,
        out_type=jax.ShapeDtypeStruct((P_DISP, DW), i32),
        mesh=plsc.VectorSubcoreMesh(**_SC_MESH),
        scratch_types=[
            pltpu.VMEM((P_DISP // NW,), i32),
            pltpu.VMEM((P_DISP // NW,), i32),
            pltpu.VMEM((P_DISP // NW, DW), i32),
            pltpu.SemaphoreType.DMA,
        ],
    )
    oext = combine(dtok, rm, yc)
    return lax.bitcast_convert_type(oext[:N_TOKENS], jnp.bfloat16).reshape(
        N_TOKENS, D_MODEL).astype(f32)


# speedup vs baseline: 3.4606x; 3.4606x over previous
import functools

import jax
import jax.numpy as jnp
from jax import lax
from jax.experimental import pallas as pl
from jax.experimental.pallas import tpu as pltpu

NUM_EXPERTS = 4
TOP_K = 2
D_MODEL = 512
D_FF = 2048
N_TOKENS = 4096
SINKHORN_ITERS = 30

def _router_body(x_ref, rw_ref, combine_ref):
    lt = lax.dot_general(
        rw_ref[...], x_ref[...],
        (((0,), (1,)), ((), ())),
        preferred_element_type=jnp.float32,
    )

    cost = jnp.exp(lt)
    n0 = jnp.float32(N_TOKENS)
    n1 = jnp.float32(NUM_EXPERTS)
    eps = jnp.float32(1e-8)

    def body(_, carry):
        d0, d1 = carry
        d0 = (1.0 / n0) / (jnp.sum(d1 * cost, axis=0, keepdims=True) + eps)
        d1 = (1.0 / n1) / (jnp.sum(d0 * cost, axis=1, keepdims=True) + eps)
        return d0, d1

    d0 = jnp.ones((1, N_TOKENS), jnp.float32)
    d1 = jnp.ones((NUM_EXPERTS, 1), jnp.float32)
    d0, d1 = lax.fori_loop(0, SINKHORN_ITERS, body, (d0, d1))
    s = d1 * cost * d0

    erow = lax.broadcasted_iota(jnp.int32, (NUM_EXPERTS, N_TOKENS), 0)

    m1 = jnp.max(s, axis=0, keepdims=True)
    i1 = jnp.min(jnp.where(s == m1, erow, NUM_EXPERTS), axis=0, keepdims=True)
    masked = jnp.where(erow == i1, float("-inf"), s)
    m2 = jnp.max(masked, axis=0, keepdims=True)
    i2 = jnp.min(jnp.where(masked == m2, erow, NUM_EXPERTS), axis=0,
                 keepdims=True)

    mx = jnp.max(lt, axis=0, keepdims=True)
    p = jnp.exp(lt - mx)
    p = p / jnp.sum(p, axis=0, keepdims=True)

    sel1 = (erow == i1).astype(jnp.float32)
    sel2 = (erow == i2).astype(jnp.float32)
    s1 = jnp.sum(p * sel1, axis=0, keepdims=True)
    s2 = jnp.sum(p * sel2, axis=0, keepdims=True)
    combine_t = s1 * sel1 + s2 * sel2

    ecol = lax.broadcasted_iota(jnp.int32, (NUM_EXPERTS, NUM_EXPERTS), 1)
    eye = (lax.broadcasted_iota(jnp.int32, (NUM_EXPERTS, NUM_EXPERTS), 0)
           == ecol).astype(jnp.float32)
    combine_ref[...] = lax.dot_general(
        combine_t, eye, (((0,), (0,)), ((), ())),
        preferred_element_type=jnp.float32,
        precision=lax.Precision.HIGHEST,
    )


def _expert_body(x_ref, w1_ref, w2_ref, combine_ref, out_ref,
                 w1b_ref, w2b_ref):
    @pl.when(pl.program_id(0) == 0)
    def _():
        for e in range(NUM_EXPERTS):
            w1b_ref[e] = w1_ref[e].astype(jnp.bfloat16)
            w2b_ref[e] = w2_ref[e].astype(jnp.bfloat16)

    xb = x_ref[...].astype(jnp.bfloat16)
    acc = jnp.zeros(out_ref.shape, jnp.float32)
    for e in range(NUM_EXPERTS):
        h = jnp.dot(xb, w1b_ref[e], preferred_element_type=jnp.float32)
        h = h * jax.nn.sigmoid(h)
        y = jnp.dot(h.astype(jnp.bfloat16), w2b_ref[e],
                    preferred_element_type=jnp.float32)
        acc = acc + y * combine_ref[:, e:e + 1]
    out_ref[...] = acc


@jax.jit
def kernel(x, router_w, w1, w2):
    combine = pl.pallas_call(
        _router_body,
        out_shape=jax.ShapeDtypeStruct((N_TOKENS, NUM_EXPERTS), jnp.float32),
    )(x, router_w)

    bt = 512
    n_t = N_TOKENS // bt
    out = pl.pallas_call(
        _expert_body,
        grid=(n_t,),
        in_specs=[
            pl.BlockSpec((bt, D_MODEL), lambda i: (i, 0)),
            pl.BlockSpec((NUM_EXPERTS, D_MODEL, D_FF), lambda i: (0, 0, 0)),
            pl.BlockSpec((NUM_EXPERTS, D_FF, D_MODEL), lambda i: (0, 0, 0)),
            pl.BlockSpec((bt, NUM_EXPERTS), lambda i: (i, 0)),
        ],
        out_specs=pl.BlockSpec((bt, D_MODEL), lambda i: (i, 0)),
        out_shape=jax.ShapeDtypeStruct((N_TOKENS, D_MODEL), jnp.float32),
        scratch_shapes=[
            pltpu.VMEM((NUM_EXPERTS, D_MODEL, D_FF), jnp.bfloat16),
            pltpu.VMEM((NUM_EXPERTS, D_FF, D_MODEL), jnp.bfloat16),
        ],
        compiler_params=pltpu.CompilerParams(
            dimension_semantics=("arbitrary",),
        ),
    )(x, w1, w2, combine)
    return out


# speedup vs baseline: 3.6102x; 1.0432x over previous
import functools

import jax
import jax.numpy as jnp
from jax import lax
from jax.experimental import pallas as pl
from jax.experimental.pallas import tpu as pltpu

NUM_EXPERTS = 4
TOP_K = 2
D_MODEL = 512
D_FF = 2048
N_TOKENS = 4096
SINKHORN_ITERS = 30

def _router_body(x_ref, rw_ref, combine_ref):
    lt = lax.dot_general(
        rw_ref[...], x_ref[...],
        (((0,), (1,)), ((), ())),
        preferred_element_type=jnp.float32,
    )

    cost = jnp.exp(lt)
    n0 = jnp.float32(N_TOKENS)
    n1 = jnp.float32(NUM_EXPERTS)
    eps = jnp.float32(1e-8)

    def body(_, carry):
        d0, d1 = carry
        d0 = (1.0 / n0) / (jnp.sum(d1 * cost, axis=0, keepdims=True) + eps)
        d1 = (1.0 / n1) / (jnp.sum(d0 * cost, axis=1, keepdims=True) + eps)
        return d0, d1

    d0 = jnp.ones((1, N_TOKENS), jnp.float32)
    d1 = jnp.ones((NUM_EXPERTS, 1), jnp.float32)
    d0, d1 = lax.fori_loop(0, SINKHORN_ITERS, body, (d0, d1))
    s = d1 * cost * d0

    erow = lax.broadcasted_iota(jnp.int32, (NUM_EXPERTS, N_TOKENS), 0)

    m1 = jnp.max(s, axis=0, keepdims=True)
    i1 = jnp.min(jnp.where(s == m1, erow, NUM_EXPERTS), axis=0, keepdims=True)
    masked = jnp.where(erow == i1, float("-inf"), s)
    m2 = jnp.max(masked, axis=0, keepdims=True)
    i2 = jnp.min(jnp.where(masked == m2, erow, NUM_EXPERTS), axis=0,
                 keepdims=True)

    mx = jnp.max(lt, axis=0, keepdims=True)
    p = jnp.exp(lt - mx)
    p = p / jnp.sum(p, axis=0, keepdims=True)

    sel1 = (erow == i1).astype(jnp.float32)
    sel2 = (erow == i2).astype(jnp.float32)
    s1 = jnp.sum(p * sel1, axis=0, keepdims=True)
    s2 = jnp.sum(p * sel2, axis=0, keepdims=True)
    combine_t = s1 * sel1 + s2 * sel2

    ecol = lax.broadcasted_iota(jnp.int32, (NUM_EXPERTS, NUM_EXPERTS), 1)
    eye = (lax.broadcasted_iota(jnp.int32, (NUM_EXPERTS, NUM_EXPERTS), 0)
           == ecol).astype(jnp.float32)
    combine_ref[...] = lax.dot_general(
        combine_t, eye, (((0,), (0,)), ((), ())),
        preferred_element_type=jnp.float32,
        precision=lax.Precision.HIGHEST,
    )


def _expert_body(x_ref, w1_ref, w2_ref, combine_ref, out_ref):
    xb = x_ref[...].astype(jnp.bfloat16)
    acc = jnp.zeros(out_ref.shape, jnp.float32)
    for e in range(NUM_EXPERTS):
        h = jnp.dot(xb, w1_ref[e].astype(jnp.bfloat16),
                    preferred_element_type=jnp.float32)
        h = h * jax.nn.sigmoid(h)
        y = jnp.dot(h.astype(jnp.bfloat16), w2_ref[e].astype(jnp.bfloat16),
                    preferred_element_type=jnp.float32)
        acc = acc + y * combine_ref[:, e:e + 1]
    out_ref[...] = acc


@jax.jit
def kernel(x, router_w, w1, w2):
    combine = pl.pallas_call(
        _router_body,
        out_shape=jax.ShapeDtypeStruct((N_TOKENS, NUM_EXPERTS), jnp.float32),
    )(x, router_w)

    bt = 1024
    n_t = N_TOKENS // bt
    out = pl.pallas_call(
        _expert_body,
        grid=(n_t,),
        in_specs=[
            pl.BlockSpec((bt, D_MODEL), lambda i: (i, 0)),
            pl.BlockSpec((NUM_EXPERTS, D_MODEL, D_FF), lambda i: (0, 0, 0)),
            pl.BlockSpec((NUM_EXPERTS, D_FF, D_MODEL), lambda i: (0, 0, 0)),
            pl.BlockSpec((bt, NUM_EXPERTS), lambda i: (i, 0)),
        ],
        out_specs=pl.BlockSpec((bt, D_MODEL), lambda i: (i, 0)),
        out_shape=jax.ShapeDtypeStruct((N_TOKENS, D_MODEL), jnp.float32),
        compiler_params=pltpu.CompilerParams(
            dimension_semantics=("arbitrary",),
        ),
    )(x, w1, w2, combine)
    return out


# speedup vs baseline: 3.6118x; 1.0004x over previous
import jax
import jax.numpy as jnp
from jax import lax
from jax.experimental import pallas as pl
from jax.experimental.pallas import tpu as pltpu

NUM_EXPERTS = 4
TOP_K = 2
D_MODEL = 512
D_FF = 2048
N_TOKENS = 4096
SINKHORN_ITERS = 30

def _router_body(x_ref, rw_ref, combine_ref):
    lt = lax.dot_general(
        rw_ref[...], x_ref[...],
        (((0,), (1,)), ((), ())),
        preferred_element_type=jnp.float32,
    )

    cost = jnp.exp(lt)
    n0 = jnp.float32(N_TOKENS)
    n1 = jnp.float32(NUM_EXPERTS)
    eps = jnp.float32(1e-8)

    def body(_, carry):
        d0, d1 = carry
        d0 = (1.0 / n0) / (jnp.sum(d1 * cost, axis=0, keepdims=True) + eps)
        d1 = (1.0 / n1) / (jnp.sum(d0 * cost, axis=1, keepdims=True) + eps)
        return d0, d1

    d0 = jnp.ones((1, N_TOKENS), jnp.float32)
    d1 = jnp.ones((NUM_EXPERTS, 1), jnp.float32)
    d0, d1 = lax.fori_loop(0, SINKHORN_ITERS, body, (d0, d1))
    s = d1 * cost * d0

    erow = lax.broadcasted_iota(jnp.int32, (NUM_EXPERTS, N_TOKENS), 0)

    m1 = jnp.max(s, axis=0, keepdims=True)
    i1 = jnp.min(jnp.where(s == m1, erow, NUM_EXPERTS), axis=0, keepdims=True)
    masked = jnp.where(erow == i1, float("-inf"), s)
    m2 = jnp.max(masked, axis=0, keepdims=True)
    i2 = jnp.min(jnp.where(masked == m2, erow, NUM_EXPERTS), axis=0,
                 keepdims=True)

    mx = jnp.max(lt, axis=0, keepdims=True)
    p = jnp.exp(lt - mx)
    p = p / jnp.sum(p, axis=0, keepdims=True)

    sel1 = (erow == i1).astype(jnp.float32)
    sel2 = (erow == i2).astype(jnp.float32)
    s1 = jnp.sum(p * sel1, axis=0, keepdims=True)
    s2 = jnp.sum(p * sel2, axis=0, keepdims=True)
    combine_t = s1 * sel1 + s2 * sel2

    ecol = lax.broadcasted_iota(jnp.int32, (NUM_EXPERTS, NUM_EXPERTS), 1)
    eye = (lax.broadcasted_iota(jnp.int32, (NUM_EXPERTS, NUM_EXPERTS), 0)
           == ecol).astype(jnp.float32)
    combine_ref[...] = lax.dot_general(
        combine_t, eye, (((0,), (0,)), ((), ())),
        preferred_element_type=jnp.float32,
        precision=lax.Precision.HIGHEST,
    )


def _expert_body(x_ref, w1_ref, w2_ref, combine_ref, out_ref):
    xb = x_ref[...].astype(jnp.bfloat16)
    acc = jnp.zeros(out_ref.shape, jnp.float32)
    for e in range(NUM_EXPERTS):
        h = jnp.dot(xb, w1_ref[e].astype(jnp.bfloat16),
                    preferred_element_type=jnp.float32)
        h = h * jax.nn.sigmoid(h)
        y = jnp.dot(h.astype(jnp.bfloat16), w2_ref[e].astype(jnp.bfloat16),
                    preferred_element_type=jnp.float32)
        acc = acc + y * combine_ref[:, e:e + 1]
    out_ref[...] = acc


@jax.jit
def kernel(x, router_w, w1, w2):
    combine = pl.pallas_call(
        _router_body,
        out_shape=jax.ShapeDtypeStruct((N_TOKENS, NUM_EXPERTS), jnp.float32),
    )(x, router_w)

    bt = 1024
    n_t = N_TOKENS // bt
    out = pl.pallas_call(
        _expert_body,
        grid=(n_t,),
        in_specs=[
            pl.BlockSpec((bt, D_MODEL), lambda i: (i, 0)),
            pl.BlockSpec((NUM_EXPERTS, D_MODEL, D_FF), lambda i: (0, 0, 0)),
            pl.BlockSpec((NUM_EXPERTS, D_FF, D_MODEL), lambda i: (0, 0, 0)),
            pl.BlockSpec((bt, NUM_EXPERTS), lambda i: (i, 0)),
        ],
        out_specs=pl.BlockSpec((bt, D_MODEL), lambda i: (i, 0)),
        out_shape=jax.ShapeDtypeStruct((N_TOKENS, D_MODEL), jnp.float32),
        compiler_params=pltpu.CompilerParams(
            dimension_semantics=("arbitrary",),
        ),
    )(x, w1, w2, combine)
    return out
